# baseline (speedup 1.0000x reference)
import jax
import jax.numpy as jnp
from jax import lax
from jax.experimental import pallas as pl
from jax.experimental.pallas import tpu as pltpu

N = 100000
D = 128
H = 64
S = 512
SSUB = 64
R = 5000
G = N // R


def _fused_kernel(x_ref, ids_ref, ids_s_ref, w1_ref, b1_ref, w2_ref,
                  out_ref, m_ref, z_ref):
    i = pl.program_id(0)
    xv = x_ref[...]
    h = jnp.dot(xv, w1_ref[...], preferred_element_type=jnp.float32)
    h = jnp.maximum(h + b1_ref[...], 0.0)
    g_col = jnp.dot(h, w2_ref[...], preferred_element_type=jnp.float32)
    g = jnp.transpose(g_col)

    @pl.when(i == 0)
    def _():
        m_ref[0, 0] = -jnp.inf
        z_ref[0, 0] = 0.0
        out_ref[...] = jnp.zeros_like(out_ref)

    m_old = m_ref[0, 0]
    m_new = jnp.maximum(m_old, jnp.max(g))
    m_ref[0, 0] = m_new
    scale = jnp.exp(m_old - m_new)

    @pl.when(jnp.logical_and(i > 0, scale < 1.0))
    def _():
        out_ref[...] *= scale

    e = jnp.exp(g - m_new)
    z_ref[0, 0] = z_ref[0, 0] * scale + jnp.sum(e)

    xb = xv.astype(jnp.bfloat16)
    ids = ids_ref[0, 0, :]
    first = ids_s_ref[0, 0, 0]
    last = ids_s_ref[0, 0, R - 1]
    base = jnp.minimum((first // 8) * 8, S - SSUB)
    fits = (last - base) < SSUB

    @pl.when(fits)
    def _():
        shifted = ids - base
        seg = lax.broadcasted_iota(jnp.int32, (SSUB, R), 0)
        ohw = jnp.where(shifted[None, :] == seg, e, 0.0).astype(jnp.bfloat16)
        contrib = jnp.dot(ohw, xb, preferred_element_type=jnp.float32)
        out_ref[pl.ds(base, SSUB), :] += contrib

    @pl.when(jnp.logical_not(fits))
    def _():
        seg = lax.broadcasted_iota(jnp.int32, (S, R), 0)
        ohw = jnp.where(ids[None, :] == seg, e, 0.0).astype(jnp.bfloat16)
        contrib = jnp.dot(ohw, xb, preferred_element_type=jnp.float32)
        out_ref[...] += contrib

    @pl.when(i == G - 1)
    def _():
        out_ref[...] *= (1.0 / z_ref[0, 0])


def kernel(x, batch, W1, b1, W2, b2):
    del b2
    ids3 = batch.astype(jnp.int32).reshape(G, 1, R)
    b1r = b1.reshape(1, H)
    out = pl.pallas_call(
        _fused_kernel,
        grid=(G,),
        in_specs=[
            pl.BlockSpec((R, D), lambda i: (i, 0)),
            pl.BlockSpec((1, 1, R), lambda i: (i, 0, 0)),
            pl.BlockSpec((1, 1, R), lambda i: (i, 0, 0), memory_space=pltpu.SMEM),
            pl.BlockSpec((D, H), lambda i: (0, 0)),
            pl.BlockSpec((1, H), lambda i: (0, 0)),
            pl.BlockSpec((H, 1), lambda i: (0, 0)),
        ],
        out_specs=pl.BlockSpec((S, D), lambda i: (0, 0)),
        out_shape=jax.ShapeDtypeStruct((S, D), jnp.float32),
        scratch_shapes=[
            pltpu.SMEM((1, 1), jnp.float32),
            pltpu.SMEM((1, 1), jnp.float32),
        ],
    )(x, ids3, ids3, W1, b1r, W2)
    return out


# speedup vs baseline: 1.3982x; 1.3982x over previous
import jax
import jax.numpy as jnp
from jax import lax
from jax.experimental import pallas as pl
from jax.experimental.pallas import tpu as pltpu

N = 100000
D = 128
H = 64
S = 512
SSUB = 128
R = 10000
G = N // R


def _fused_kernel(x_ref, ids_ref, ids_s_ref, w1_ref, b1_ref, w2_ref,
                  out_ref, m_ref, z_ref):
    i = pl.program_id(0)
    xv = x_ref[...]
    h = jnp.dot(xv, w1_ref[...], preferred_element_type=jnp.float32)
    h = jnp.maximum(h + b1_ref[...], 0.0)
    g_col = jnp.dot(h, w2_ref[...], preferred_element_type=jnp.float32)
    g = jnp.transpose(g_col)

    @pl.when(i == 0)
    def _():
        m_ref[0, 0] = -jnp.inf
        z_ref[0, 0] = 0.0
        out_ref[...] = jnp.zeros_like(out_ref)

    m_old = m_ref[0, 0]
    m_new = jnp.maximum(m_old, jnp.max(g))
    m_ref[0, 0] = m_new
    scale = jnp.exp(m_old - m_new)

    @pl.when(jnp.logical_and(i > 0, scale < 1.0))
    def _():
        out_ref[...] *= scale

    e = jnp.exp(g - m_new)
    z_ref[0, 0] = z_ref[0, 0] * scale + jnp.sum(e)

    xb = xv.astype(jnp.bfloat16)
    ids = ids_ref[0, 0, :]
    first = ids_s_ref[0, 0, 0]
    last = ids_s_ref[0, 0, R - 1]
    base = jnp.minimum((first // 8) * 8, S - SSUB)
    fits = (last - base) < SSUB

    @pl.when(fits)
    def _():
        shifted = ids - base
        seg = lax.broadcasted_iota(jnp.int32, (SSUB, R), 0)
        ohw = jnp.where(shifted[None, :] == seg, e, 0.0).astype(jnp.bfloat16)
        contrib = jnp.dot(ohw, xb, preferred_element_type=jnp.float32)
        out_ref[pl.ds(base, SSUB), :] += contrib

    @pl.when(jnp.logical_not(fits))
    def _():
        seg = lax.broadcasted_iota(jnp.int32, (S, R), 0)
        ohw = jnp.where(ids[None, :] == seg, e, 0.0).astype(jnp.bfloat16)
        contrib = jnp.dot(ohw, xb, preferred_element_type=jnp.float32)
        out_ref[...] += contrib

    @pl.when(i == G - 1)
    def _():
        out_ref[...] *= (1.0 / z_ref[0, 0])


def kernel(x, batch, W1, b1, W2, b2):
    del b2
    ids3 = batch.astype(jnp.int32).reshape(G, 1, R)
    b1r = b1.reshape(1, H)
    out = pl.pallas_call(
        _fused_kernel,
        grid=(G,),
        in_specs=[
            pl.BlockSpec((R, D), lambda i: (i, 0)),
            pl.BlockSpec((1, 1, R), lambda i: (i, 0, 0)),
            pl.BlockSpec((1, 1, R), lambda i: (i, 0, 0), memory_space=pltpu.SMEM),
            pl.BlockSpec((D, H), lambda i: (0, 0)),
            pl.BlockSpec((1, H), lambda i: (0, 0)),
            pl.BlockSpec((H, 1), lambda i: (0, 0)),
        ],
        out_specs=pl.BlockSpec((S, D), lambda i: (0, 0)),
        out_shape=jax.ShapeDtypeStruct((S, D), jnp.float32),
        scratch_shapes=[
            pltpu.SMEM((1, 1), jnp.float32),
            pltpu.SMEM((1, 1), jnp.float32),
        ],
    )(x, ids3, ids3, W1, b1r, W2)
    return out


# speedup vs baseline: 1.4725x; 1.0532x over previous
import jax
import jax.numpy as jnp
from jax import lax
from jax.experimental import pallas as pl
from jax.experimental.pallas import tpu as pltpu

N = 100000
D = 128
H = 64
S = 512
SSUB = 128
R = 20000
G = N // R


def _fused_kernel(x_ref, ids_ref, ids_s_ref, w1_ref, b1_ref, w2_ref,
                  out_ref, m_ref, z_ref):
    i = pl.program_id(0)
    xv = x_ref[...]
    h = jnp.dot(xv, w1_ref[...], preferred_element_type=jnp.float32)
    h = jnp.maximum(h + b1_ref[...], 0.0)
    g_col = jnp.dot(h, w2_ref[...], preferred_element_type=jnp.float32)
    g = jnp.transpose(g_col)

    @pl.when(i == 0)
    def _():
        m_ref[0, 0] = -jnp.inf
        z_ref[0, 0] = 0.0
        out_ref[...] = jnp.zeros_like(out_ref)

    m_old = m_ref[0, 0]
    m_new = jnp.maximum(m_old, jnp.max(g))
    m_ref[0, 0] = m_new
    scale = jnp.exp(m_old - m_new)

    @pl.when(jnp.logical_and(i > 0, scale < 1.0))
    def _():
        out_ref[...] *= scale

    e = jnp.exp(g - m_new)
    z_ref[0, 0] = z_ref[0, 0] * scale + jnp.sum(e)

    xb = xv.astype(jnp.bfloat16)
    ids = ids_ref[0, 0, :]
    first = ids_s_ref[0, 0, 0]
    last = ids_s_ref[0, 0, R - 1]
    base = jnp.minimum((first // 8) * 8, S - SSUB)
    fits = (last - base) < SSUB

    @pl.when(fits)
    def _():
        shifted = ids - base
        seg = lax.broadcasted_iota(jnp.int32, (SSUB, R), 0)
        ohw = jnp.where(shifted[None, :] == seg, e, 0.0).astype(jnp.bfloat16)
        contrib = jnp.dot(ohw, xb, preferred_element_type=jnp.float32)
        out_ref[pl.ds(base, SSUB), :] += contrib

    @pl.when(jnp.logical_not(fits))
    def _():
        for c in range(S // SSUB):
            cbase = c * SSUB
            seg = lax.broadcasted_iota(jnp.int32, (SSUB, R), 0) + cbase
            ohw = jnp.where(ids[None, :] == seg, e, 0.0).astype(jnp.bfloat16)
            contrib = jnp.dot(ohw, xb, preferred_element_type=jnp.float32)
            out_ref[pl.ds(cbase, SSUB), :] += contrib

    @pl.when(i == G - 1)
    def _():
        out_ref[...] *= (1.0 / z_ref[0, 0])


def kernel(x, batch, W1, b1, W2, b2):
    del b2
    ids3 = batch.astype(jnp.int32).reshape(G, 1, R)
    b1r = b1.reshape(1, H)
    out = pl.pallas_call(
        _fused_kernel,
        grid=(G,),
        in_specs=[
            pl.BlockSpec((R, D), lambda i: (i, 0)),
            pl.BlockSpec((1, 1, R), lambda i: (i, 0, 0)),
            pl.BlockSpec((1, 1, R), lambda i: (i, 0, 0), memory_space=pltpu.SMEM),
            pl.BlockSpec((D, H), lambda i: (0, 0)),
            pl.BlockSpec((1, H), lambda i: (0, 0)),
            pl.BlockSpec((H, 1), lambda i: (0, 0)),
        ],
        out_specs=pl.BlockSpec((S, D), lambda i: (0, 0)),
        out_shape=jax.ShapeDtypeStruct((S, D), jnp.float32),
        scratch_shapes=[
            pltpu.SMEM((1, 1), jnp.float32),
            pltpu.SMEM((1, 1), jnp.float32),
        ],
    )(x, ids3, ids3, W1, b1r, W2)
    return out


# speedup vs baseline: 1.5510x; 1.0533x over previous
import jax
import jax.numpy as jnp
from jax import lax
from jax.experimental import pallas as pl
from jax.experimental.pallas import tpu as pltpu

N = 100000
D = 128
H = 64
S = 512
SSUB = 128
R = 20000
G = N // R


def _fused_kernel(x_ref, ids_ref, ids_s_ref, w1_ref, b1_ref, w2_ref,
                  out_ref, m_ref, z_ref):
    i = pl.program_id(0)
    xv = x_ref[...]
    xb = xv.astype(jnp.bfloat16)
    h = jnp.dot(xb, w1_ref[...], preferred_element_type=jnp.float32)
    h = jnp.maximum(h + b1_ref[...], 0.0)
    g_col = jnp.dot(h.astype(jnp.bfloat16), w2_ref[...],
                    preferred_element_type=jnp.float32)
    g = jnp.transpose(g_col)

    @pl.when(i == 0)
    def _():
        m_ref[0, 0] = -jnp.inf
        z_ref[0, 0] = 0.0
        out_ref[...] = jnp.zeros_like(out_ref)

    m_old = m_ref[0, 0]
    m_new = jnp.maximum(m_old, jnp.max(g))
    m_ref[0, 0] = m_new
    scale = jnp.exp(m_old - m_new)

    @pl.when(jnp.logical_and(i > 0, scale < 1.0))
    def _():
        out_ref[...] *= scale

    e = jnp.exp(g - m_new)
    z_ref[0, 0] = z_ref[0, 0] * scale + jnp.sum(e)

    ids = ids_ref[0, 0, :]
    first = ids_s_ref[0, 0, 0]
    last = ids_s_ref[0, 0, R - 1]
    base = jnp.minimum((first // 8) * 8, S - SSUB)
    fits = (last - base) < SSUB

    @pl.when(fits)
    def _():
        shifted = ids - base
        seg = lax.broadcasted_iota(jnp.int32, (SSUB, R), 0)
        ohw = jnp.where(shifted[None, :] == seg, e, 0.0).astype(jnp.bfloat16)
        contrib = jnp.dot(ohw, xb, preferred_element_type=jnp.float32)
        out_ref[pl.ds(base, SSUB), :] += contrib

    @pl.when(jnp.logical_not(fits))
    def _():
        for c in range(S // SSUB):
            cbase = c * SSUB
            seg = lax.broadcasted_iota(jnp.int32, (SSUB, R), 0) + cbase
            ohw = jnp.where(ids[None, :] == seg, e, 0.0).astype(jnp.bfloat16)
            contrib = jnp.dot(ohw, xb, preferred_element_type=jnp.float32)
            out_ref[pl.ds(cbase, SSUB), :] += contrib

    @pl.when(i == G - 1)
    def _():
        out_ref[...] *= (1.0 / z_ref[0, 0])


def kernel(x, batch, W1, b1, W2, b2):
    del b2
    ids3 = batch.astype(jnp.int32).reshape(G, 1, R)
    b1r = b1.reshape(1, H)
    W1 = W1.astype(jnp.bfloat16)
    W2 = W2.astype(jnp.bfloat16)
    out = pl.pallas_call(
        _fused_kernel,
        grid=(G,),
        in_specs=[
            pl.BlockSpec((R, D), lambda i: (i, 0)),
            pl.BlockSpec((1, 1, R), lambda i: (i, 0, 0)),
            pl.BlockSpec((1, 1, R), lambda i: (i, 0, 0), memory_space=pltpu.SMEM),
            pl.BlockSpec((D, H), lambda i: (0, 0)),
            pl.BlockSpec((1, H), lambda i: (0, 0)),
            pl.BlockSpec((H, 1), lambda i: (0, 0)),
        ],
        out_specs=pl.BlockSpec((S, D), lambda i: (0, 0)),
        out_shape=jax.ShapeDtypeStruct((S, D), jnp.float32),
        scratch_shapes=[
            pltpu.SMEM((1, 1), jnp.float32),
            pltpu.SMEM((1, 1), jnp.float32),
        ],
    )(x, ids3, ids3, W1, b1r, W2)
    return out
